# baseline (speedup 1.0000x reference)
import functools

import jax
import jax.numpy as jnp
from jax import lax
from jax.experimental import pallas as pl
from jax.experimental.pallas import tpu as pltpu, tpu_sc as plsc

N_NODES = 10000
N_EDGES = 320000
D_IN = 128
D_HID = 128
D_OUT = 16
WEIGHT_DECAY = 5e-4

NC = 2
NS = 16
NW = NC * NS
E_PAD = 327680
EPW = E_PAD // NW
CHUNK = 64
NCHUNK = EPW // CHUNK
CPS = 32
NSUPER = NCHUNK // CPS
N_PAD = 10240
ROWS_PER_SUB = N_PAD // NS

NGRP = 4
GW = D_OUT // NGRP
NBLK = NC * NGRP
BLK_CH = E_PAD // NBLK // CHUNK
NSUP2 = BLK_CH // CPS


NROWS = E_PAD // CHUNK
U0 = 6
U1 = 4


@functools.lru_cache(maxsize=None)
def _make_spmm128():
    mesh = plsc.VectorSubcoreMesh(core_axis_name="c", subcore_axis_name="s")

    @functools.partial(
        pl.kernel,
        out_type=jax.ShapeDtypeStruct((NC * N_PAD, D_HID), jnp.float32),
        mesh=mesh,
        scratch_types=[
            pltpu.VMEM_SHARED((N_PAD, D_HID), jnp.float32),
            pltpu.VMEM((CPS, CHUNK), jnp.int32),
            pltpu.VMEM((CPS, CHUNK), jnp.int32),
            pltpu.VMEM((CPS, CHUNK), jnp.float32),
            pltpu.VMEM((CHUNK, D_HID), jnp.float32),
            pltpu.VMEM((CHUNK, D_HID), jnp.float32),
            pltpu.SemaphoreType.DMA,
            pltpu.SemaphoreType.DMA,
        ],
    )
    def spmm(h_hbm, src_hbm, dst_hbm, w_hbm, zero_hbm, out_hbm,
             acc, src_v, dst_v, w_v, rows0_v, rows1_v, sem0, sem1):
        c = lax.axis_index("c")
        s = lax.axis_index("s")
        rs = pl.ds(s * ROWS_PER_SUB, ROWS_PER_SUB)
        pltpu.sync_copy(zero_hbm.at[rs], acc.at[rs])
        plsc.subcore_barrier()

        nsup = jnp.where(c == 0, U0, U1)
        row_base = jnp.where(c == 0, s * (U0 * CPS),
                             NS * U0 * CPS + s * (U1 * CPS))

        def scale(rows_v, k):
            for g in range(CHUNK // 16):
                wvec = w_v[k, pl.ds(g * 16, 16)]
                for j in range(16):
                    e = g * 16 + j
                    wsc = wvec[j]
                    for f in range(D_HID // 16):
                        rows_v[e, pl.ds(f * 16, 16)] = (
                            rows_v[e, pl.ds(f * 16, 16)] * wsc)

        def super_body(u, carry):
            rows = pl.ds(row_base + u * CPS, CPS)
            pltpu.sync_copy(src_hbm.at[rows], src_v)
            pltpu.sync_copy(dst_hbm.at[rows], dst_v)
            pltpu.sync_copy(w_hbm.at[rows], w_v)
            pltpu.async_copy(h_hbm.at[src_v.at[0]], rows0_v, sem0)

            def pair_body(j, carry2):
                k0 = 2 * j
                k1 = 2 * j + 1
                cp1 = pltpu.async_copy(h_hbm.at[src_v.at[k1]], rows1_v, sem1)
                pltpu.make_async_copy(h_hbm.at[src_v.at[k0]], rows0_v,
                                      sem0).wait()
                scale(rows0_v, k0)
                pltpu.sync_copy(rows0_v, acc.at[dst_v.at[k0]], add=True)

                @pl.when(j + 1 < CPS // 2)
                def _():
                    pltpu.async_copy(h_hbm.at[src_v.at[k0 + 2]], rows0_v,
                                     sem0)

                cp1.wait()
                scale(rows1_v, k1)
                pltpu.sync_copy(rows1_v, acc.at[dst_v.at[k1]], add=True)
                return carry2

            lax.fori_loop(0, CPS // 2, pair_body, carry)
            return carry

        lax.fori_loop(0, nsup, super_body, 0)
        plsc.subcore_barrier()
        pltpu.sync_copy(acc.at[rs],
                        out_hbm.at[pl.ds(c * N_PAD + s * ROWS_PER_SUB,
                                         ROWS_PER_SUB)])

    return spmm


def _vgather(vec, idx):
    dn = lax.GatherDimensionNumbers(
        offset_dims=(), collapsed_slice_dims=(0,), start_index_map=(0,))
    return lax.gather(vec, idx[:, None], dn, (1,),
                      mode=lax.GatherScatterMode.PROMISE_IN_BOUNDS)


@functools.lru_cache(maxsize=None)
def _make_spmm16():
    mesh = plsc.VectorSubcoreMesh(core_axis_name="c", subcore_axis_name="s")

    @functools.partial(
        pl.kernel,
        out_type=jax.ShapeDtypeStruct((NW, N_PAD * GW), jnp.float32),
        mesh=mesh,
        scratch_types=[
            pltpu.VMEM((N_PAD * GW,), jnp.float32),
            pltpu.VMEM((N_PAD * GW,), jnp.float32),
            pltpu.VMEM((CPS * CHUNK,), jnp.int32),
            pltpu.VMEM((CPS * CHUNK,), jnp.int32),
            pltpu.VMEM((CPS * CHUNK,), jnp.float32),
        ],
        compiler_params=pltpu.CompilerParams(needs_layout_passes=False),
    )
    def spmm16(h2g_hbm, src_hbm, dst_hbm, w_hbm, zero_hbm, out_hbm,
               tab_v, acc_v, src_v, dst_v, w_v):
        c = lax.axis_index("c")
        s = lax.axis_index("s")
        wid = s * NC + c
        g = s // NGRP
        b = (s % NGRP) * NC + c
        pltpu.sync_copy(h2g_hbm.at[g], tab_v)
        pltpu.sync_copy(zero_hbm, acc_v)

        lane = lax.iota(jnp.int32, 16)
        rep4 = lane // GW
        mod4 = lane - rep4 * GW
        masks = [rep4 == i for i in range(4)]

        def super_body(u, carry):
            base = (b * BLK_CH + u * CPS) * CHUNK
            pltpu.sync_copy(src_hbm.at[pl.ds(base, CPS * CHUNK)], src_v)
            pltpu.sync_copy(dst_hbm.at[pl.ds(base, CPS * CHUNK)], dst_v)
            pltpu.sync_copy(w_hbm.at[pl.ds(base, CPS * CHUNK)], w_v)

            def chunk_body(k, carry2):
                for q in range(CHUNK // 16):
                    srcv = src_v[pl.ds(k * CHUNK + q * 16, 16)]
                    dstv = dst_v[pl.ds(k * CHUNK + q * 16, 16)]
                    wv = w_v[pl.ds(k * CHUNK + q * 16, 16)]
                    for t in range(4):
                        qidx = t * 4 + rep4
                        srcq = _vgather(srcv, qidx)
                        dstq = _vgather(dstv, qidx)
                        wq = _vgather(wv, qidx)
                        vals = plsc.load_gather(
                            tab_v, [srcq * GW + mod4]) * wq
                        didx = dstq * GW + mod4
                        for i in range(4):
                            plsc.addupdate_scatter(acc_v, [didx],
                                                   vals, mask=masks[i])
                return carry2

            lax.fori_loop(0, CPS, chunk_body, carry)
            return carry

        lax.fori_loop(0, NSUP2, super_body, 0)
        pltpu.sync_copy(acc_v, out_hbm.at[wid])

    return spmm16


def _mm1_body(x_ref, w_ref, o_ref):
    o_ref[...] = jnp.dot(x_ref[...], w_ref[...],
                         preferred_element_type=jnp.float32)


def _mm2_body(p0_ref, p1_ref, w_ref, o_ref):
    h = jnp.maximum(p0_ref[...] + p1_ref[...], 0.0)
    o_ref[...] = jnp.dot(h, w_ref[...], preferred_element_type=jnp.float32)


def _red_body(o_ref, r_ref):
    xp = o_ref[...]
    r_ref[...] = jnp.concatenate(
        [jnp.sum(xp[NBLK * g:NBLK * (g + 1)], axis=0, keepdims=True)
         for g in range(NGRP)],
        axis=0)


def _loss_body(o_ref, lab_ref, m_ref, w1_ref, loss_ref, acc_ref):
    out = o_ref[...]
    lab = lab_ref[...]
    m = m_ref[...]
    mx = jnp.max(out, axis=-1, keepdims=True)
    sh = out - mx
    lse = jnp.log(jnp.sum(jnp.exp(sh), axis=-1, keepdims=True))
    ce = -jnp.sum(lab * (sh - lse), axis=-1, keepdims=True)
    iota = lax.broadcasted_iota(jnp.int32, (N_PAD, D_OUT), 1)
    ao = jnp.min(jnp.where(out == mx, iota, D_OUT), axis=-1, keepdims=True)
    lmx = jnp.max(lab, axis=-1, keepdims=True)
    al = jnp.min(jnp.where(lab == lmx, iota, D_OUT), axis=-1, keepdims=True)
    correct = (ao == al).astype(jnp.float32)
    msum = jnp.sum(m)
    wd = WEIGHT_DECAY * 0.5 * jnp.sum(w1_ref[...] * w1_ref[...])
    loss_ref[...] = jnp.reshape(wd + jnp.sum(ce * m) / msum, (1, 1))
    acc_ref[...] = jnp.reshape(jnp.sum(correct * m) / msum, (1, 1))


def kernel(x, label, mask, edge_index, edge_weight, W1, W2):
    src = edge_index[0].astype(jnp.int32)
    dst = edge_index[1].astype(jnp.int32)
    pad = E_PAD - N_EDGES
    spread = jnp.arange(pad, dtype=jnp.int32) % N_NODES
    src = jnp.concatenate([src, spread])
    dst = jnp.concatenate([dst, spread])
    w = jnp.concatenate([edge_weight, jnp.zeros((pad,), jnp.float32)])
    src2d = src.reshape(NROWS, CHUNK)
    dst2d = dst.reshape(NROWS, CHUNK)
    w2d = w.reshape(NROWS, CHUNK)
    zero128 = jnp.zeros((N_PAD, D_HID), jnp.float32)
    zero16 = jnp.zeros((N_PAD * GW,), jnp.float32)

    h = pl.pallas_call(
        _mm1_body,
        grid=(10,),
        in_specs=[pl.BlockSpec((1000, D_IN), lambda i: (i, 0)),
                  pl.BlockSpec((D_IN, D_HID), lambda i: (0, 0))],
        out_specs=pl.BlockSpec((1000, D_HID), lambda i: (i, 0)),
        out_shape=jax.ShapeDtypeStruct((N_NODES, D_HID), jnp.float32),
    )(x, W1)

    p = _make_spmm128()(h, src2d, dst2d, w2d, zero128)
    p = p.reshape(NC, N_PAD, D_HID)

    h2 = pl.pallas_call(
        _mm2_body,
        grid=(10,),
        in_specs=[pl.BlockSpec((1024, D_HID), lambda i: (i, 0)),
                  pl.BlockSpec((1024, D_HID), lambda i: (i, 0)),
                  pl.BlockSpec((D_HID, D_OUT), lambda i: (0, 0))],
        out_specs=pl.BlockSpec((1024, D_OUT), lambda i: (i, 0)),
        out_shape=jax.ShapeDtypeStruct((N_PAD, D_OUT), jnp.float32),
    )(p[0], p[1], W2)

    h2g = h2.reshape(N_PAD, NGRP, GW).transpose(1, 0, 2).reshape(
        NGRP, N_PAD * GW)
    o = _make_spmm16()(h2g, src, dst, w, zero16)

    red = pl.pallas_call(
        _red_body,
        in_specs=[pl.BlockSpec((NW, N_PAD * GW), lambda: (0, 0))],
        out_specs=pl.BlockSpec((NGRP, N_PAD * GW), lambda: (0, 0)),
        out_shape=jax.ShapeDtypeStruct((NGRP, N_PAD * GW), jnp.float32),
    )(o)
    outf = red.reshape(NGRP, N_PAD, GW).transpose(1, 0, 2).reshape(
        N_PAD, D_OUT)

    labp = jnp.pad(label, ((0, N_PAD - N_NODES), (0, 0)))
    maskf = jnp.pad(mask.astype(jnp.float32), (0, N_PAD - N_NODES))
    maskf = maskf.reshape(N_PAD, 1)
    loss, acc = pl.pallas_call(
        _loss_body,
        in_specs=[pl.BlockSpec((N_PAD, D_OUT), lambda: (0, 0)),
                  pl.BlockSpec((N_PAD, D_OUT), lambda: (0, 0)),
                  pl.BlockSpec((N_PAD, 1), lambda: (0, 0)),
                  pl.BlockSpec((D_IN, D_HID), lambda: (0, 0))],
        out_specs=[pl.BlockSpec((1, 1), lambda: (0, 0)),
                   pl.BlockSpec((1, 1), lambda: (0, 0))],
        out_shape=[jax.ShapeDtypeStruct((1, 1), jnp.float32),
                   jax.ShapeDtypeStruct((1, 1), jnp.float32)],
    )(outf, labp, maskf, W1)
    return (loss[0, 0], acc[0, 0])


# speedup vs baseline: 1.0625x; 1.0625x over previous
import functools

import jax
import jax.numpy as jnp
from jax import lax
from jax.experimental import pallas as pl
from jax.experimental.pallas import tpu as pltpu, tpu_sc as plsc

N_NODES = 10000
N_EDGES = 320000
D_IN = 128
D_HID = 128
D_OUT = 16
WEIGHT_DECAY = 5e-4

NC = 2
NS = 16
NW = NC * NS
E_PAD = 327680
EPW = E_PAD // NW
CHUNK = 64
NCHUNK = EPW // CHUNK
CPS = 32
NSUPER = NCHUNK // CPS
N_PAD = 10240
ROWS_PER_SUB = N_PAD // NS

NGRP = 4
GW = D_OUT // NGRP
NBLK = NC * NGRP
BLK_CH = E_PAD // NBLK // CHUNK
NSUP2 = BLK_CH // CPS


NROWS = E_PAD // CHUNK
U0 = 5
U1 = 5


@functools.lru_cache(maxsize=None)
def _make_spmm128():
    mesh = plsc.VectorSubcoreMesh(core_axis_name="c", subcore_axis_name="s")

    @functools.partial(
        pl.kernel,
        out_type=jax.ShapeDtypeStruct((NC * N_PAD, D_HID), jnp.float32),
        mesh=mesh,
        scratch_types=[
            pltpu.VMEM_SHARED((N_PAD, D_HID), jnp.float32),
            pltpu.VMEM((CPS, CHUNK), jnp.int32),
            pltpu.VMEM((CPS, CHUNK), jnp.int32),
            pltpu.VMEM((CPS, CHUNK), jnp.float32),
            pltpu.VMEM((CHUNK, D_HID), jnp.float32),
            pltpu.VMEM((CHUNK, D_HID), jnp.float32),
            pltpu.SemaphoreType.DMA,
            pltpu.SemaphoreType.DMA,
        ],
    )
    def spmm(h_hbm, src_hbm, dst_hbm, w_hbm, zero_hbm, out_hbm,
             acc, src_v, dst_v, w_v, rows0_v, rows1_v, sem0, sem1):
        c = lax.axis_index("c")
        s = lax.axis_index("s")
        rs = pl.ds(s * ROWS_PER_SUB, ROWS_PER_SUB)
        pltpu.sync_copy(zero_hbm.at[rs], acc.at[rs])
        plsc.subcore_barrier()

        nsup = jnp.where(c == 0, U0, U1)
        row_base = jnp.where(c == 0, s * (U0 * CPS),
                             NS * U0 * CPS + s * (U1 * CPS))

        def scale(rows_v, k):
            for g in range(CHUNK // 16):
                wvec = w_v[k, pl.ds(g * 16, 16)]
                for j in range(16):
                    e = g * 16 + j
                    wsc = wvec[j]
                    for f in range(D_HID // 16):
                        rows_v[e, pl.ds(f * 16, 16)] = (
                            rows_v[e, pl.ds(f * 16, 16)] * wsc)

        def super_body(u, carry):
            rows = pl.ds(row_base + u * CPS, CPS)
            pltpu.sync_copy(src_hbm.at[rows], src_v)
            pltpu.sync_copy(dst_hbm.at[rows], dst_v)
            pltpu.sync_copy(w_hbm.at[rows], w_v)
            pltpu.async_copy(h_hbm.at[src_v.at[0]], rows0_v, sem0)

            def pair_body(j, carry2):
                k0 = 2 * j
                k1 = 2 * j + 1
                cp1 = pltpu.async_copy(h_hbm.at[src_v.at[k1]], rows1_v, sem1)
                pltpu.make_async_copy(h_hbm.at[src_v.at[k0]], rows0_v,
                                      sem0).wait()
                scale(rows0_v, k0)
                pltpu.sync_copy(rows0_v, acc.at[dst_v.at[k0]], add=True)

                @pl.when(j + 1 < CPS // 2)
                def _():
                    pltpu.async_copy(h_hbm.at[src_v.at[k0 + 2]], rows0_v,
                                     sem0)

                cp1.wait()
                scale(rows1_v, k1)
                pltpu.sync_copy(rows1_v, acc.at[dst_v.at[k1]], add=True)
                return carry2

            lax.fori_loop(0, CPS // 2, pair_body, carry)
            return carry

        lax.fori_loop(0, nsup, super_body, 0)
        plsc.subcore_barrier()
        pltpu.sync_copy(acc.at[rs],
                        out_hbm.at[pl.ds(c * N_PAD + s * ROWS_PER_SUB,
                                         ROWS_PER_SUB)])

    return spmm


def _vgather(vec, idx):
    dn = lax.GatherDimensionNumbers(
        offset_dims=(), collapsed_slice_dims=(0,), start_index_map=(0,))
    return lax.gather(vec, idx[:, None], dn, (1,),
                      mode=lax.GatherScatterMode.PROMISE_IN_BOUNDS)


@functools.lru_cache(maxsize=None)
def _make_spmm16():
    mesh = plsc.VectorSubcoreMesh(core_axis_name="c", subcore_axis_name="s")

    @functools.partial(
        pl.kernel,
        out_type=jax.ShapeDtypeStruct((NW, N_PAD * GW), jnp.float32),
        mesh=mesh,
        scratch_types=[
            pltpu.VMEM((N_PAD * GW,), jnp.float32),
            pltpu.VMEM((N_PAD * GW,), jnp.float32),
            pltpu.VMEM((CPS * CHUNK,), jnp.int32),
            pltpu.VMEM((CPS * CHUNK,), jnp.int32),
            pltpu.VMEM((CPS * CHUNK,), jnp.float32),
        ],
        compiler_params=pltpu.CompilerParams(needs_layout_passes=False),
    )
    def spmm16(h2g_hbm, src_hbm, dst_hbm, w_hbm, zero_hbm, out_hbm,
               tab_v, acc_v, src_v, dst_v, w_v):
        c = lax.axis_index("c")
        s = lax.axis_index("s")
        wid = s * NC + c
        g = s // NGRP
        b = (s % NGRP) * NC + c
        pltpu.sync_copy(h2g_hbm.at[g], tab_v)
        pltpu.sync_copy(zero_hbm, acc_v)

        lane = lax.iota(jnp.int32, 16)
        rep4 = lane // GW
        mod4 = lane - rep4 * GW
        masks = [rep4 == i for i in range(4)]

        def super_body(u, carry):
            base = (b * BLK_CH + u * CPS) * CHUNK
            pltpu.sync_copy(src_hbm.at[pl.ds(base, CPS * CHUNK)], src_v)
            pltpu.sync_copy(dst_hbm.at[pl.ds(base, CPS * CHUNK)], dst_v)
            pltpu.sync_copy(w_hbm.at[pl.ds(base, CPS * CHUNK)], w_v)

            def chunk_body(k, carry2):
                for q in range(CHUNK // 16):
                    srcv = src_v[pl.ds(k * CHUNK + q * 16, 16)]
                    dstv = dst_v[pl.ds(k * CHUNK + q * 16, 16)]
                    wv = w_v[pl.ds(k * CHUNK + q * 16, 16)]
                    for t in range(4):
                        qidx = t * 4 + rep4
                        srcq = _vgather(srcv, qidx)
                        dstq = _vgather(dstv, qidx)
                        wq = _vgather(wv, qidx)
                        vals = plsc.load_gather(
                            tab_v, [srcq * GW + mod4]) * wq
                        didx = dstq * GW + mod4
                        for i in range(4):
                            plsc.addupdate_scatter(acc_v, [didx],
                                                   vals, mask=masks[i])
                return carry2

            lax.fori_loop(0, CPS, chunk_body, carry)
            return carry

        lax.fori_loop(0, NSUP2, super_body, 0)
        pltpu.sync_copy(acc_v, out_hbm.at[wid])

    return spmm16


def _mm1_body(x_ref, w_ref, o_ref):
    o_ref[...] = jnp.dot(x_ref[...], w_ref[...],
                         preferred_element_type=jnp.float32)


def _mm2_body(p0_ref, p1_ref, w_ref, o_ref):
    h = jnp.maximum(p0_ref[...] + p1_ref[...], 0.0)
    o_ref[...] = jnp.dot(h, w_ref[...], preferred_element_type=jnp.float32)


def _red_body(o_ref, r_ref):
    xp = o_ref[...]
    r_ref[...] = jnp.concatenate(
        [jnp.sum(xp[NBLK * g:NBLK * (g + 1)], axis=0, keepdims=True)
         for g in range(NGRP)],
        axis=0)


def _loss_body(o_ref, lab_ref, m_ref, w1_ref, loss_ref, acc_ref):
    out = o_ref[...]
    lab = lab_ref[...]
    m = m_ref[...]
    mx = jnp.max(out, axis=-1, keepdims=True)
    sh = out - mx
    lse = jnp.log(jnp.sum(jnp.exp(sh), axis=-1, keepdims=True))
    ce = -jnp.sum(lab * (sh - lse), axis=-1, keepdims=True)
    iota = lax.broadcasted_iota(jnp.int32, (N_PAD, D_OUT), 1)
    ao = jnp.min(jnp.where(out == mx, iota, D_OUT), axis=-1, keepdims=True)
    lmx = jnp.max(lab, axis=-1, keepdims=True)
    al = jnp.min(jnp.where(lab == lmx, iota, D_OUT), axis=-1, keepdims=True)
    correct = (ao == al).astype(jnp.float32)
    msum = jnp.sum(m)
    wd = WEIGHT_DECAY * 0.5 * jnp.sum(w1_ref[...] * w1_ref[...])
    loss_ref[...] = jnp.reshape(wd + jnp.sum(ce * m) / msum, (1, 1))
    acc_ref[...] = jnp.reshape(jnp.sum(correct * m) / msum, (1, 1))


def kernel(x, label, mask, edge_index, edge_weight, W1, W2):
    src = edge_index[0].astype(jnp.int32)
    dst = edge_index[1].astype(jnp.int32)
    pad = E_PAD - N_EDGES
    spread = jnp.arange(pad, dtype=jnp.int32) % N_NODES
    src = jnp.concatenate([src, spread])
    dst = jnp.concatenate([dst, spread])
    w = jnp.concatenate([edge_weight, jnp.zeros((pad,), jnp.float32)])
    src2d = src.reshape(NROWS, CHUNK)
    dst2d = dst.reshape(NROWS, CHUNK)
    w2d = w.reshape(NROWS, CHUNK)
    zero128 = jnp.zeros((N_PAD, D_HID), jnp.float32)
    zero16 = jnp.zeros((N_PAD * GW,), jnp.float32)

    h = pl.pallas_call(
        _mm1_body,
        grid=(10,),
        in_specs=[pl.BlockSpec((1000, D_IN), lambda i: (i, 0)),
                  pl.BlockSpec((D_IN, D_HID), lambda i: (0, 0))],
        out_specs=pl.BlockSpec((1000, D_HID), lambda i: (i, 0)),
        out_shape=jax.ShapeDtypeStruct((N_NODES, D_HID), jnp.float32),
    )(x, W1)

    p = _make_spmm128()(h, src2d, dst2d, w2d, zero128)
    p = p.reshape(NC, N_PAD, D_HID)

    h2 = pl.pallas_call(
        _mm2_body,
        grid=(10,),
        in_specs=[pl.BlockSpec((1024, D_HID), lambda i: (i, 0)),
                  pl.BlockSpec((1024, D_HID), lambda i: (i, 0)),
                  pl.BlockSpec((D_HID, D_OUT), lambda i: (0, 0))],
        out_specs=pl.BlockSpec((1024, D_OUT), lambda i: (i, 0)),
        out_shape=jax.ShapeDtypeStruct((N_PAD, D_OUT), jnp.float32),
    )(p[0], p[1], W2)

    h2g = h2.reshape(N_PAD, NGRP, GW).transpose(1, 0, 2).reshape(
        NGRP, N_PAD * GW)
    o = _make_spmm16()(h2g, src, dst, w, zero16)

    red = pl.pallas_call(
        _red_body,
        in_specs=[pl.BlockSpec((NW, N_PAD * GW), lambda: (0, 0))],
        out_specs=pl.BlockSpec((NGRP, N_PAD * GW), lambda: (0, 0)),
        out_shape=jax.ShapeDtypeStruct((NGRP, N_PAD * GW), jnp.float32),
    )(o)
    outf = red.reshape(NGRP, N_PAD, GW).transpose(1, 0, 2).reshape(
        N_PAD, D_OUT)

    labp = jnp.pad(label, ((0, N_PAD - N_NODES), (0, 0)))
    maskf = jnp.pad(mask.astype(jnp.float32), (0, N_PAD - N_NODES))
    maskf = maskf.reshape(N_PAD, 1)
    loss, acc = pl.pallas_call(
        _loss_body,
        in_specs=[pl.BlockSpec((N_PAD, D_OUT), lambda: (0, 0)),
                  pl.BlockSpec((N_PAD, D_OUT), lambda: (0, 0)),
                  pl.BlockSpec((N_PAD, 1), lambda: (0, 0)),
                  pl.BlockSpec((D_IN, D_HID), lambda: (0, 0))],
        out_specs=[pl.BlockSpec((1, 1), lambda: (0, 0)),
                   pl.BlockSpec((1, 1), lambda: (0, 0))],
        out_shape=[jax.ShapeDtypeStruct((1, 1), jnp.float32),
                   jax.ShapeDtypeStruct((1, 1), jnp.float32)],
    )(outf, labp, maskf, W1)
    return (loss[0, 0], acc[0, 0])
